# baseline (speedup 1.0000x reference)
import jax
import jax.numpy as jnp
from jax.experimental import pallas as pl
from jax.experimental.pallas import tpu as pltpu

_E = 8
_DG = 1024
_DM = 1024
_DO = 64
_N = 16384
_T = 2048
_GRID = _N // _T


def _router_body(gf_ref, wg_ref, bg_ref, probs_ref, assign_ref, topkp_ref):
    logits = jnp.dot(gf_ref[...], wg_ref[...],
                     preferred_element_type=jnp.float32) + bg_ref[...]
    lt = logits.T
    m = jnp.max(lt, axis=0, keepdims=True)
    ex = jnp.exp(lt - m)
    inv_z = 1.0 / jnp.sum(ex, axis=0, keepdims=True)
    sub = jax.lax.broadcasted_iota(jnp.int32, lt.shape, 0)
    amax_t = jnp.min(jnp.where(lt == m, sub, _E), axis=0, keepdims=True)

    probs_ref[...] = ex * inv_z
    assign_ref[...] = amax_t
    topkp_ref[...] = inv_z


def _expert_body(x_ref, assign_ref, wall_ref, loss_ref,
                 sums_ref, counts_ref):
    step = pl.program_id(0)

    @pl.when(step == 0)
    def _init():
        sums_ref[...] = jnp.zeros_like(sums_ref)
        counts_ref[...] = jnp.zeros_like(counts_ref)

    h = jnp.dot(x_ref[...].astype(jnp.bfloat16), wall_ref[...],
                preferred_element_type=jnp.float32)
    p2 = h * h
    r0 = jax.lax.broadcasted_iota(jnp.int32, (_E * _DO, _E), 0) // _DO
    c0 = jax.lax.broadcasted_iota(jnp.int32, (_E * _DO, _E), 1)
    sel = jnp.where(r0 == c0, jnp.float32(1.0 / _DO), jnp.float32(0.0))
    per_all_t = jnp.dot(p2, sel, preferred_element_type=jnp.float32).T

    sub = jax.lax.broadcasted_iota(jnp.int32, (_E, _T), 0)
    onehot = (sub == assign_ref[...]).astype(jnp.float32)
    sums_ref[...] += jnp.sum(onehot * per_all_t, axis=1, keepdims=True)
    counts_ref[...] += jnp.sum(onehot, axis=1, keepdims=True)

    @pl.when(step == _GRID - 1)
    def _fini():
        cnt = counts_ref[...]
        loss_e = sums_ref[...] / jnp.maximum(cnt, 1.0)
        loss_ref[...] = jnp.sum(jnp.where(cnt > 0, loss_e, 0.0),
                                axis=0, keepdims=True)


def kernel(gate_features, x, Wg, bg, W_experts):
    wall = W_experts.transpose(1, 0, 2).reshape(_DM, _E * _DO)
    wall = wall.astype(jnp.bfloat16)
    bg2 = bg.reshape(1, _E)

    probs_t, assign_t, topkp_t = pl.pallas_call(
        _router_body,
        grid=(_GRID,),
        in_specs=[
            pl.BlockSpec((_T, _DG), lambda i: (i, 0)),
            pl.BlockSpec((_DG, _E), lambda i: (0, 0)),
            pl.BlockSpec((1, _E), lambda i: (0, 0)),
        ],
        out_specs=[
            pl.BlockSpec((_E, _T), lambda i: (0, i)),
            pl.BlockSpec((1, _T), lambda i: (0, i)),
            pl.BlockSpec((1, _T), lambda i: (0, i)),
        ],
        out_shape=[
            jax.ShapeDtypeStruct((_E, _N), jnp.float32),
            jax.ShapeDtypeStruct((1, _N), jnp.int32),
            jax.ShapeDtypeStruct((1, _N), jnp.float32),
        ],
    )(gate_features, Wg, bg2)

    loss = pl.pallas_call(
        _expert_body,
        grid=(_GRID,),
        in_specs=[
            pl.BlockSpec((_T, _DM), lambda i: (i, 0)),
            pl.BlockSpec((1, _T), lambda i: (0, i)),
            pl.BlockSpec((_DM, _E * _DO), lambda i: (0, 0)),
        ],
        out_specs=pl.BlockSpec((1, 1), lambda i: (0, 0)),
        out_shape=jax.ShapeDtypeStruct((1, 1), jnp.float32),
        scratch_shapes=[
            pltpu.VMEM((_E, 1), jnp.float32),
            pltpu.VMEM((_E, 1), jnp.float32),
        ],
    )(x, assign_t, wall)

    assign = assign_t.reshape(_N)
    return (loss.reshape(()), assign, probs_t.T,
            assign.reshape(_N, 1), topkp_t.reshape(_N, 1))
